# baseline (speedup 1.0000x reference)
import math

import jax
import jax.numpy as jnp
import numpy as np
from jax import lax
from jax.experimental import pallas as pl
from jax.experimental.pallas import tpu as pltpu

PATCH_LEN = 16
STRIDE = 8
SEQ_LEN = 512
D_MODEL = 512
HIDDEN = 256
PATCH_NUM = 64
NWIN = 505
NPAD = 512
XPAD = 528


def _pos_emb(n, d_model):
    position = np.arange(n, dtype=np.float32)[:, None]
    div_term = np.exp(np.arange(0, d_model, 2, dtype=np.float32) * -(math.log(10000.0) / d_model))
    pe = np.zeros((n, d_model), dtype=np.float32)
    pe[:, 0::2] = np.sin(position * div_term)
    pe[:, 1::2] = np.cos(position * div_term)
    return pe


def _srs_body(x_ref, w1t_ref, b1c_ref, w2t_ref, b2c_ref,
              w1s_ref, b1s_ref, w2s_ref, b2s_ref,
              wreg_ref, wirr_ref, alpha_ref, pe_ref, out_ref):
    f32 = jnp.float32
    xb = x_ref[0]

    xr_t = jnp.concatenate([xb[:, p:p + NPAD] for p in range(PATCH_LEN)], axis=0)

    pre1 = jnp.dot(w1t_ref[...], xr_t, preferred_element_type=f32)
    hid = jnp.maximum(pre1 + b1c_ref[...], 0.0)
    scores = jnp.dot(w2t_ref[...], hid, preferred_element_type=f32) + b2c_ref[...]

    iota_n = lax.broadcasted_iota(jnp.int32, (PATCH_NUM, NPAD), 1)
    scores = jnp.where(iota_n < NWIN, scores, -1e30)

    maxv = jnp.max(scores, axis=1, keepdims=True)
    eq = scores == maxv
    idx = jnp.min(jnp.where(eq, iota_n, NPAD), axis=1, keepdims=True)
    ms_ind = (maxv != 0.0).astype(f32)

    oh = jnp.where(iota_n == idx, ms_ind, 0.0)
    dnums_nt = (((1,), (1,)), ((), ()))
    selected = lax.dot_general(oh, xr_t, dnums_nt, preferred_element_type=f32)

    iota_j = lax.broadcasted_iota(jnp.int32, (PATCH_NUM, NPAD), 0)
    e_org = (iota_n == 8 * iota_j).astype(f32)
    origin = lax.dot_general(e_org, xr_t, dnums_nt, preferred_element_type=f32)

    hid_s = jnp.maximum(jnp.dot(selected, w1s_ref[...], preferred_element_type=f32)
                        + b1s_ref[...], 0.0)
    sh_row = lax.dot_general(w2s_ref[...], hid_s, (((0,), (1,)), ((), ())),
                             preferred_element_type=f32) + b2s_ref[...]
    ident = (lax.broadcasted_iota(jnp.int32, (PATCH_NUM, PATCH_NUM), 0)
             == lax.broadcasted_iota(jnp.int32, (PATCH_NUM, PATCH_NUM), 1)).astype(f32)
    sh_col = lax.dot_general(ident, sh_row, dnums_nt, preferred_element_type=f32)

    iota_jj = lax.broadcasted_iota(jnp.int32, (PATCH_NUM, PATCH_NUM), 0)
    iota_ii = lax.broadcasted_iota(jnp.int32, (PATCH_NUM, PATCH_NUM), 1)
    lt = sh_col < sh_row
    eq2 = sh_col == sh_row
    cmp2 = jnp.where(lt | (eq2 & (iota_jj < iota_ii)), 1.0, 0.0)
    rank_row = jnp.sum(cmp2, axis=0, keepdims=True)

    iota_k = lax.broadcasted_iota(jnp.int32, (PATCH_NUM, PATCH_NUM), 0)
    nz_row = (sh_row != 0.0).astype(f32)
    perm = jnp.where(rank_row == iota_k.astype(f32), nz_row, 0.0)
    shuffled = jnp.dot(perm, selected, preferred_element_type=f32)

    a = alpha_ref[0, 0]
    w = 1.0 / (1.0 + jnp.exp(-a))
    emb = (w * jnp.dot(origin, wreg_ref[...], preferred_element_type=f32)
           + (1.0 - w) * jnp.dot(shuffled, wirr_ref[...], preferred_element_type=f32)
           + pe_ref[...])
    out_ref[0] = emb


def kernel(x, W1_sel, b1_sel, W2_sel, b2_sel, W1_shf, b1_shf, W2_shf, b2_shf,
           W_reg, W_irr, alpha):
    b, c, L = x.shape
    S = b * c
    xf = x.reshape(S, L)
    x_pad = jnp.concatenate([xf, jnp.repeat(xf[:, -1:], XPAD - L, axis=1)], axis=1)
    x_pad = x_pad.reshape(S, 1, XPAD)

    pe = jnp.asarray(_pos_emb(PATCH_NUM, D_MODEL))

    args = (
        x_pad,
        W1_sel.T,
        b1_sel.reshape(HIDDEN, 1),
        W2_sel.T,
        b2_sel.reshape(PATCH_NUM, 1),
        W1_shf,
        b1_shf.reshape(1, HIDDEN),
        W2_shf,
        b2_shf.reshape(1, 1),
        W_reg,
        W_irr,
        alpha.reshape(1, 1),
        pe,
    )

    def const_spec(shape):
        return pl.BlockSpec(shape, lambda i: (0,) * len(shape))

    in_specs = [
        pl.BlockSpec((1, 1, XPAD), lambda i: (i, 0, 0)),
        const_spec((HIDDEN, PATCH_LEN)),
        const_spec((HIDDEN, 1)),
        const_spec((PATCH_NUM, HIDDEN)),
        const_spec((PATCH_NUM, 1)),
        const_spec((PATCH_LEN, HIDDEN)),
        const_spec((1, HIDDEN)),
        const_spec((HIDDEN, 1)),
        const_spec((1, 1)),
        const_spec((PATCH_LEN, D_MODEL)),
        const_spec((PATCH_LEN, D_MODEL)),
        const_spec((1, 1)),
        const_spec((PATCH_NUM, D_MODEL)),
    ]

    out = pl.pallas_call(
        _srs_body,
        grid=(S,),
        in_specs=in_specs,
        out_specs=pl.BlockSpec((1, PATCH_NUM, D_MODEL), lambda i: (i, 0, 0)),
        out_shape=jax.ShapeDtypeStruct((S, PATCH_NUM, D_MODEL), jnp.float32),
        compiler_params=pltpu.CompilerParams(
            dimension_semantics=("arbitrary",),
        ),
    )(*args)
    return out


# speedup vs baseline: 2.8193x; 2.8193x over previous
import math

import jax
import jax.numpy as jnp
import numpy as np
from jax import lax
from jax.experimental import pallas as pl
from jax.experimental.pallas import tpu as pltpu

PATCH_LEN = 16
STRIDE = 8
SEQ_LEN = 512
D_MODEL = 512
HIDDEN = 256
PATCH_NUM = 64
NWIN = 505
SEG = 640
SB = 8
MLANE = SB * SEG


def _pos_emb(n, d_model):
    position = np.arange(n, dtype=np.float32)[:, None]
    div_term = np.exp(np.arange(0, d_model, 2, dtype=np.float32) * -(math.log(10000.0) / d_model))
    pe = np.zeros((n, d_model), dtype=np.float32)
    pe[:, 0::2] = np.sin(position * div_term)
    pe[:, 1::2] = np.cos(position * div_term)
    return pe


NT = (((1,), (1,)), ((), ()))


def _srs_body(x_ref, w1t_ref, b1c_ref, w2t_ref, b2c_ref,
              w1s_ref, b1s_ref, w2s_ref, b2s_ref,
              wreg_ref, wirr_ref, alpha_ref, pe_ref, out_ref):
    f32 = jnp.float32
    xflat = x_ref[...]

    pieces = [xflat]
    for p in range(1, PATCH_LEN):
        pieces.append(jnp.concatenate([xflat[:, p:], xflat[:, :p]], axis=1))
    xr_all = jnp.concatenate(pieces, axis=0)

    pre1 = jnp.dot(w1t_ref[...], xr_all, preferred_element_type=f32)
    hid = jnp.maximum(pre1 + b1c_ref[...], 0.0)

    scr_rows = []
    for s in range(SB):
        h_s = hid[:, s * SEG:(s + 1) * SEG]
        scr_rows.append(jnp.dot(w2t_ref[...], h_s, preferred_element_type=f32)
                        + b2c_ref[...])
    scr = jnp.concatenate(scr_rows, axis=0)

    iota_u = lax.broadcasted_iota(jnp.int32, (SB * PATCH_NUM, SEG), 1)
    scr = jnp.where(iota_u < NWIN, scr, -1e30)

    maxv = jnp.max(scr, axis=1, keepdims=True)
    eq = scr == maxv
    idx = jnp.min(jnp.where(eq, iota_u, SEG), axis=1, keepdims=True)
    ms_ind = (maxv != 0.0).astype(f32)
    oh = jnp.where(iota_u == idx, ms_ind, 0.0)

    iota_j = lax.broadcasted_iota(jnp.int32, (PATCH_NUM, SEG), 0)
    iota_u1 = lax.broadcasted_iota(jnp.int32, (PATCH_NUM, SEG), 1)
    e_org = (iota_u1 == 8 * iota_j).astype(f32)

    sel_rows, org_rows = [], []
    for s in range(SB):
        xr_s = xr_all[:, s * SEG:(s + 1) * SEG]
        oh_s = oh[s * PATCH_NUM:(s + 1) * PATCH_NUM]
        sel_rows.append(lax.dot_general(oh_s, xr_s, NT, preferred_element_type=f32,
                                        precision=lax.Precision.HIGHEST))
        org_rows.append(lax.dot_general(e_org, xr_s, NT, preferred_element_type=f32))
    sel = jnp.concatenate(sel_rows, axis=0)
    org = jnp.concatenate(org_rows, axis=0)

    hid_s = jnp.maximum(jnp.dot(sel, w1s_ref[...], preferred_element_type=f32)
                        + b1s_ref[...], 0.0)
    shc = jnp.dot(hid_s, w2s_ref[...], preferred_element_type=f32) + b2s_ref[...]

    iota_jj = lax.broadcasted_iota(jnp.int32, (PATCH_NUM, PATCH_NUM), 0)
    iota_ii = lax.broadcasted_iota(jnp.int32, (PATCH_NUM, PATCH_NUM), 1)
    iota_k = iota_jj.astype(f32)

    shuf_rows = []
    for s in range(SB):
        sh_col = shc[s * PATCH_NUM:(s + 1) * PATCH_NUM]
        sh_row = jnp.swapaxes(sh_col, 0, 1)
        lt = sh_col < sh_row
        eq2 = sh_col == sh_row
        cmp2 = jnp.where(lt | (eq2 & (iota_jj < iota_ii)), 1.0, 0.0)
        rank_row = jnp.sum(cmp2, axis=0, keepdims=True)
        nz_row = (sh_row != 0.0).astype(f32)
        perm = jnp.where(rank_row == iota_k, nz_row, 0.0)
        shuf_rows.append(jnp.dot(perm, sel[s * PATCH_NUM:(s + 1) * PATCH_NUM],
                                 preferred_element_type=f32))
    shuf = jnp.concatenate(shuf_rows, axis=0)

    a = alpha_ref[0, 0]
    w = 1.0 / (1.0 + jnp.exp(-a))
    emb = (w * jnp.dot(org, wreg_ref[...], preferred_element_type=f32)
           + (1.0 - w) * jnp.dot(shuf, wirr_ref[...], preferred_element_type=f32))
    out_ref[...] = emb.reshape(SB, PATCH_NUM, D_MODEL) + pe_ref[...][None]


def kernel(x, W1_sel, b1_sel, W2_sel, b2_sel, W1_shf, b1_shf, W2_shf, b2_shf,
           W_reg, W_irr, alpha):
    b, c, L = x.shape
    S = b * c
    xf = x.reshape(S, L)
    x_pad = jnp.concatenate(
        [xf, jnp.repeat(xf[:, -1:], SEG - L, axis=1)], axis=1)
    x_flat = x_pad.reshape(1, S * SEG)

    pe = jnp.asarray(_pos_emb(PATCH_NUM, D_MODEL))

    args = (
        x_flat,
        W1_sel.T,
        b1_sel.reshape(HIDDEN, 1),
        W2_sel.T,
        b2_sel.reshape(PATCH_NUM, 1),
        W1_shf,
        b1_shf.reshape(1, HIDDEN),
        W2_shf,
        b2_shf.reshape(1, 1),
        W_reg,
        W_irr,
        alpha.reshape(1, 1),
        pe,
    )

    def const_spec(shape):
        return pl.BlockSpec(shape, lambda i: (0,) * len(shape))

    in_specs = [
        pl.BlockSpec((1, MLANE), lambda i: (0, i)),
        const_spec((HIDDEN, PATCH_LEN)),
        const_spec((HIDDEN, 1)),
        const_spec((PATCH_NUM, HIDDEN)),
        const_spec((PATCH_NUM, 1)),
        const_spec((PATCH_LEN, HIDDEN)),
        const_spec((1, HIDDEN)),
        const_spec((HIDDEN, 1)),
        const_spec((1, 1)),
        const_spec((PATCH_LEN, D_MODEL)),
        const_spec((PATCH_LEN, D_MODEL)),
        const_spec((1, 1)),
        const_spec((PATCH_NUM, D_MODEL)),
    ]

    out = pl.pallas_call(
        _srs_body,
        grid=(S // SB,),
        in_specs=in_specs,
        out_specs=pl.BlockSpec((SB, PATCH_NUM, D_MODEL), lambda i: (i, 0, 0)),
        out_shape=jax.ShapeDtypeStruct((S, PATCH_NUM, D_MODEL), jnp.float32),
        compiler_params=pltpu.CompilerParams(
            dimension_semantics=("arbitrary",),
        ),
    )(*args)
    return out
